# baseline (speedup 1.0000x reference)
import functools

import jax
import jax.numpy as jnp
from jax import lax
from jax.experimental import pallas as pl
from jax.experimental.pallas import tpu as pltpu
from jax.experimental.pallas import tpu_sc as plsc

N_USERS = 100000
N_ITEMS = 100000
K = 128
B = 4096
U_MEAN = 3.5

NC = 2
NS = 16
L = 16
NW = NC * NS
BPW = B // NW

_mesh = plsc.VectorSubcoreMesh(core_axis_name="c", subcore_axis_name="s")


@functools.partial(
    pl.kernel,
    mesh=_mesh,
    out_type=jax.ShapeDtypeStruct((B,), jnp.float32),
    scratch_types=[
        pltpu.VMEM((BPW,), jnp.int32),
        pltpu.VMEM((BPW,), jnp.int32),
        pltpu.VMEM((BPW,), jnp.float32),
        pltpu.VMEM((BPW,), jnp.float32),
        pltpu.VMEM((BPW,), jnp.float32),
        pltpu.VMEM((BPW, K), jnp.float32),
        pltpu.VMEM((BPW, K), jnp.int32),
        pltpu.VMEM((BPW, K), jnp.float32),
        pltpu.VMEM((BPW,), jnp.float32),
        pltpu.SemaphoreType.DMA,
        pltpu.SemaphoreType.DMA,
    ],
)
def _svdpp(u_h, i_h, bu_h, bi_h, pu_h, qif_h, ru_h, out_h,
           u_v, i_v, bu_v, bi_v, ru_v, pu_v, qidx_v, qv_v, res_v,
           sem_a, sem_b):
    wid = lax.axis_index("s") * NC + lax.axis_index("c")
    base = wid * BPW

    pltpu.sync_copy(u_h.at[pl.ds(base, BPW)], u_v)
    pltpu.sync_copy(i_h.at[pl.ds(base, BPW)], i_v)

    cp_pu = pltpu.async_copy(pu_h.at[u_v], pu_v, sem_a)
    cp_bu = pltpu.async_copy(bu_h.at[u_v], bu_v, sem_a)
    cp_bi = pltpu.async_copy(bi_h.at[i_v], bi_v, sem_a)
    cp_ru = pltpu.async_copy(ru_h.at[u_v], ru_v, sem_a)

    kofs = [lax.iota(jnp.int32, L) * N_ITEMS + c * L * N_ITEMS
            for c in range(K // L)]

    def gen_fire(cc, carry):
        iv = i_v[pl.ds(cc * L, L)]
        for jj in range(L):
            j = cc * L + jj
            ibc = lax.broadcast(iv[jj], (L,))
            for c in range(K // L):
                qidx_v[j, pl.ds(c * L, L)] = ibc + kofs[c]
            pltpu.async_copy(qif_h.at[qidx_v.at[j]], qv_v.at[j], sem_b)
        return carry

    lax.fori_loop(0, BPW // L, gen_fire, 0)

    cp_pu.wait()
    cp_bu.wait()
    cp_bi.wait()
    cp_ru.wait()

    def drain(j, carry):
        pltpu.make_async_copy(qif_h.at[qidx_v.at[j]], qv_v.at[j], sem_b).wait()
        return carry

    lax.fori_loop(0, BPW, drain, 0)

    lane = lax.iota(jnp.int32, L)
    zero = jnp.zeros((L,), jnp.float32)

    def group_body(g, carry):
        sl = pl.ds(g * L, L)
        ruv = ru_v[sl]
        acc = zero
        for jj in range(L):
            j = g * L + jj
            rbc = lax.broadcast(ruv[jj], (L,))
            pa = zero
            for c in range(K // L):
                csl = pl.ds(c * L, L)
                pa = pa + (pu_v[j, csl] + rbc) * qv_v[j, csl]
            s01 = pa[0] + pa[1]
            s23 = pa[2] + pa[3]
            s45 = pa[4] + pa[5]
            s67 = pa[6] + pa[7]
            s89 = pa[8] + pa[9]
            sab = pa[10] + pa[11]
            scd = pa[12] + pa[13]
            sef = pa[14] + pa[15]
            s = ((s01 + s23) + (s45 + s67)) + ((s89 + sab) + (scd + sef))
            acc = jnp.where(lane == jj, lax.broadcast(s, (L,)), acc)
        res_v[sl] = bu_v[sl] + bi_v[sl] + U_MEAN + acc
        return carry

    lax.fori_loop(0, BPW // L, group_body, 0)

    pltpu.sync_copy(res_v, out_h.at[pl.ds(base, BPW)])


def kernel(u, i, bu, bi, pu, qi, Ru):
    return _svdpp(
        u.astype(jnp.int32),
        i.astype(jnp.int32),
        bu,
        bi,
        pu,
        qi.reshape(-1),
        Ru.reshape(-1),
    )
